# baseline (speedup 1.0000x reference)
import functools

import jax
import jax.numpy as jnp
from jax import lax
from jax.experimental import pallas as pl
from jax.experimental.pallas import tpu as pltpu
from jax.experimental.pallas import tpu_sc as plsc

L = 16
NC = 2
NS = 16
NW = NC * NS
ROWS = 1024
COLS = 72 * 72
ROWS_PER_W = ROWS // NW
CHUNKS = COLS // L
UNROLL = 4
NEG_THRESHOLD = 0.01
K = 3


def _gather16(x, idx):
    return x.at[idx].get(mode="promise_in_bounds")


def _xlane_max(x, perms):
    for p in perms:
        x = jnp.maximum(x, _gather16(x, p))
    return x


def _xlane_min_i32(x, perms):
    for p in perms:
        x = jnp.minimum(x, _gather16(x, p))
    return x


def _sc_body(p_hbm, l_hbm, out_hbm, p_v, l_v, o_v, sem):
    c = lax.axis_index("c")
    s = lax.axis_index("s")
    wid = s * NC + c
    base = wid * ROWS_PER_W

    lane = lax.broadcasted_iota(jnp.int32, (L,), 0)
    perms = [lane ^ (1 << k) for k in range(4)]
    zero = jnp.zeros((L,), jnp.float32)

    def row_step(i, acc):
        row = base + i
        cp_p = pltpu.make_async_copy(p_hbm.at[pl.ds(row * COLS, COLS)], p_v, sem)
        cp_l = pltpu.make_async_copy(l_hbm.at[pl.ds(row * COLS, COLS)], l_v, sem)
        cp_p.start()
        cp_l.start()
        cp_p.wait()
        cp_l.wait()

        def chunk_step(j, carry):
            t1, t2, t3 = carry
            for u in range(UNROLL):
                off = (j * UNROLL + u) * L
                pv = p_v[pl.ds(off, L)]
                lv = l_v[pl.ds(off, L)]
                v = jnp.where(lv < NEG_THRESHOLD, jnp.maximum(pv, 0.0), zero)
                m1 = jnp.minimum(t1, v)
                t1 = jnp.maximum(t1, v)
                m2 = jnp.minimum(t2, m1)
                t2 = jnp.maximum(t2, m1)
                t3 = jnp.maximum(t3, m2)
            return (t1, t2, t3)

        t1, t2, t3 = lax.fori_loop(0, CHUNKS // UNROLL, chunk_step,
                                   (zero, zero, zero))

        for _ in range(K):
            g = _xlane_max(t1, perms)
            acc = acc + g
            idx = jnp.where(t1 == g, lane, L)
            mi = _xlane_min_i32(idx, perms)
            first = lane == mi
            t1 = jnp.where(first, t2, t1)
            t2 = jnp.where(first, t3, t2)
            t3 = jnp.where(first, 0.0, t3)
        return acc

    acc = lax.fori_loop(0, ROWS_PER_W, row_step, zero)
    o_v[...] = acc
    pltpu.sync_copy(o_v, out_hbm.at[pl.ds(wid * L, L)])


@jax.jit
def _distractor_loss(p_flat, l_flat):
    mesh = plsc.VectorSubcoreMesh(core_axis_name="c", subcore_axis_name="s")
    partials = pl.kernel(
        _sc_body,
        mesh=mesh,
        out_type=jax.ShapeDtypeStruct((NW * L,), jnp.float32),
        scratch_types=[
            pltpu.VMEM((COLS,), jnp.float32),
            pltpu.VMEM((COLS,), jnp.float32),
            pltpu.VMEM((L,), jnp.float32),
            pltpu.SemaphoreType.DMA,
        ],
    )(p_flat, l_flat)
    return partials.reshape(NW, L)[:, 0].sum() / (ROWS * K)


def kernel(prediction, label):
    p_flat = prediction.reshape(-1)
    l_flat = label.reshape(-1)
    return _distractor_loss(p_flat, l_flat)


# speedup vs baseline: 1.1793x; 1.1793x over previous
import functools

import jax
import jax.numpy as jnp
from jax import lax
from jax.experimental import pallas as pl
from jax.experimental.pallas import tpu as pltpu
from jax.experimental.pallas import tpu_sc as plsc

L = 16
NC = 2
NS = 16
NW = NC * NS
ROWS = 1024
COLS = 72 * 72
ROWS_PER_W = ROWS // NW
CHUNKS = COLS // L
UNROLL = 12
B_ROWS = 4
NB = ROWS_PER_W // B_ROWS
NEG_THRESHOLD = 0.01
K = 3


def _gather16(x, idx):
    return x.at[idx].get(mode="promise_in_bounds")


def _xlane_max(x, perms):
    for p in perms:
        x = jnp.maximum(x, _gather16(x, p))
    return x


def _xlane_min_i32(x, perms):
    for p in perms:
        x = jnp.minimum(x, _gather16(x, p))
    return x


def _sc_body(p_hbm, l_hbm, out_hbm, p0, l0, p1, l1, o_v, sem0, sem1):
    c = lax.axis_index("c")
    s = lax.axis_index("s")
    wid = s * NC + c
    base = wid * ROWS_PER_W

    lane = lax.broadcasted_iota(jnp.int32, (L,), 0)
    perms = [lane ^ (1 << k) for k in range(4)]
    zero = jnp.zeros((L,), jnp.float32)
    bufs = ((p0, l0, sem0), (p1, l1, sem1))

    def batch_copies(k, bufset):
        pb, lb, sem = bufset
        off = (base + k * B_ROWS) * COLS
        n = B_ROWS * COLS
        return (pltpu.make_async_copy(p_hbm.at[pl.ds(off, n)], pb, sem),
                pltpu.make_async_copy(l_hbm.at[pl.ds(off, n)], lb, sem))

    def start_batch(k, bufset):
        for cp in batch_copies(k, bufset):
            cp.start()

    def wait_batch(k, bufset):
        for cp in batch_copies(k, bufset):
            cp.wait()

    def process_batch(pb, lb, acc):
        for r in range(B_ROWS):
            row_off = r * COLS

            def chunk_step(j, carry):
                t1, t2, t3 = carry
                for u in range(UNROLL):
                    off = row_off + (j * UNROLL + u) * L
                    pv = pb[pl.ds(off, L)]
                    lv = lb[pl.ds(off, L)]
                    v = jnp.where(lv < NEG_THRESHOLD,
                                  jnp.maximum(pv, 0.0), zero)
                    m1 = jnp.minimum(t1, v)
                    t1 = jnp.maximum(t1, v)
                    m2 = jnp.minimum(t2, m1)
                    t2 = jnp.maximum(t2, m1)
                    t3 = jnp.maximum(t3, m2)
                return (t1, t2, t3)

            t1, t2, t3 = lax.fori_loop(0, CHUNKS // UNROLL, chunk_step,
                                       (zero, zero, zero))

            for k in range(K):
                g = _xlane_max(t1, perms)
                acc = acc + g
                if k == K - 1:
                    break
                idx = jnp.where(t1 == g, lane, L)
                mi = _xlane_min_i32(idx, perms)
                first = lane == mi
                t1 = jnp.where(first, t2, t1)
                t2 = jnp.where(first, t3, t2)
        return acc

    start_batch(0, bufs[0])

    def body(b, acc):
        k0 = 2 * b
        wait_batch(k0, bufs[0])
        start_batch(k0 + 1, bufs[1])
        acc = process_batch(p0, l0, acc)
        wait_batch(k0 + 1, bufs[1])

        @pl.when(k0 + 2 < NB)
        def _():
            start_batch(k0 + 2, bufs[0])

        acc = process_batch(p1, l1, acc)
        return acc

    acc = lax.fori_loop(0, NB // 2, body, zero)
    o_v[...] = acc
    pltpu.sync_copy(o_v, out_hbm.at[pl.ds(wid * L, L)])


@jax.jit
def _distractor_loss(p_flat, l_flat):
    mesh = plsc.VectorSubcoreMesh(core_axis_name="c", subcore_axis_name="s")
    partials = pl.kernel(
        _sc_body,
        mesh=mesh,
        out_type=jax.ShapeDtypeStruct((NW * L,), jnp.float32),
        scratch_types=[
            pltpu.VMEM((B_ROWS * COLS,), jnp.float32),
            pltpu.VMEM((B_ROWS * COLS,), jnp.float32),
            pltpu.VMEM((B_ROWS * COLS,), jnp.float32),
            pltpu.VMEM((B_ROWS * COLS,), jnp.float32),
            pltpu.VMEM((L,), jnp.float32),
            pltpu.SemaphoreType.DMA,
            pltpu.SemaphoreType.DMA,
        ],
    )(p_flat, l_flat)
    return partials.reshape(NW, L)[:, 0].sum() / (ROWS * K)


def kernel(prediction, label):
    p_flat = prediction.reshape(-1)
    l_flat = label.reshape(-1)
    return _distractor_loss(p_flat, l_flat)


# speedup vs baseline: 1.7104x; 1.4504x over previous
import functools

import jax
import jax.numpy as jnp
from jax import lax
from jax.experimental import pallas as pl
from jax.experimental.pallas import tpu as pltpu
from jax.experimental.pallas import tpu_sc as plsc

L = 16
NC = 2
NS = 16
NW = NC * NS
IMGS = 1024
W = 72
IMGS_PER_W = IMGS // NW
B_IMG = 2
NB = IMGS_PER_W // B_IMG
ROWS_PER_STEP = 4
NEG_THRESHOLD = 0.01
K = 3


def _merge3(a, b):
    a1, a2, a3 = a
    b1, b2, b3 = b
    r1 = jnp.maximum(a1, b1)
    r2 = jnp.maximum(jnp.minimum(a1, b1), jnp.maximum(a2, b2))
    r3 = jnp.maximum(jnp.maximum(a3, b3),
                     jnp.maximum(jnp.minimum(a2, b1), jnp.minimum(a1, b2)))
    return r1, r2, r3


def _gather16(x, idx):
    return x.at[idx].get(mode="promise_in_bounds")


def _xlane_max(x, perms):
    for p in perms:
        x = jnp.maximum(x, _gather16(x, p))
    return x


def _xlane_min_i32(x, perms):
    for p in perms:
        x = jnp.minimum(x, _gather16(x, p))
    return x


def _sc_body(p_hbm, l_hbm, out_hbm, p0, l0, p1, l1, o_v, sem0, sem1):
    c = lax.axis_index("c")
    s = lax.axis_index("s")
    wid = s * NC + c
    base = wid * IMGS_PER_W

    lane = lax.broadcasted_iota(jnp.int32, (L,), 0)
    perms = [lane ^ (1 << k) for k in range(4)]
    zero = jnp.zeros((L,), jnp.float32)
    hi = lane >= 8
    bufs = ((p0, l0, sem0), (p1, l1, sem1))

    def batch_copies(k, bufset):
        pb, lb, sem = bufset
        i0 = base + k * B_IMG
        return (pltpu.make_async_copy(p_hbm.at[pl.ds(i0, B_IMG)], pb, sem),
                pltpu.make_async_copy(l_hbm.at[pl.ds(i0, B_IMG)], lb, sem))

    def start_batch(k, bufset):
        for cp in batch_copies(k, bufset):
            cp.start()

    def wait_batch(k, bufset):
        for cp in batch_copies(k, bufset):
            cp.wait()

    def process_batch(pb, lb, acc):
        for i in range(B_IMG):

            def row_step(j, carry):
                sets = [list(carry[3 * rr:3 * rr + 3])
                        for rr in range(ROWS_PER_STEP)]
                for u in range(5):
                    off = 16 * u if u < 4 else 56
                    for rr in range(ROWS_PER_STEP):
                        r = j * ROWS_PER_STEP + rr
                        t1, t2, t3 = sets[rr]
                        pv = pb[i, r, pl.ds(off, L)]
                        lv = lb[i, r, pl.ds(off, L)]
                        m = lv < NEG_THRESHOLD
                        if u == 4:
                            m = jnp.logical_and(m, hi)
                        v = jnp.where(m, pv, zero)
                        m1 = jnp.minimum(t1, v)
                        t1 = jnp.maximum(t1, v)
                        m2 = jnp.minimum(t2, m1)
                        t2 = jnp.maximum(t2, m1)
                        t3 = jnp.maximum(t3, m2)
                        sets[rr] = [t1, t2, t3]
                return tuple(x for s in sets for x in s)

            carry = lax.fori_loop(0, W // ROWS_PER_STEP, row_step,
                                  (zero,) * (3 * ROWS_PER_STEP))
            sets = [tuple(carry[3 * rr:3 * rr + 3])
                    for rr in range(ROWS_PER_STEP)]
            while len(sets) > 1:
                sets = [_merge3(sets[k], sets[k + 1])
                        for k in range(0, len(sets), 2)]
            t1, t2, t3 = sets[0]

            for k in range(K):
                g = _xlane_max(t1, perms)
                acc = acc + g
                if k == K - 1:
                    break
                idx = jnp.where(t1 == g, lane, L)
                mi = _xlane_min_i32(idx, perms)
                first = lane == mi
                t1 = jnp.where(first, t2, t1)
                t2 = jnp.where(first, t3, t2)
        return acc

    start_batch(0, bufs[0])

    def body(b, acc):
        k0 = 2 * b
        wait_batch(k0, bufs[0])
        start_batch(k0 + 1, bufs[1])
        acc = process_batch(p0, l0, acc)
        wait_batch(k0 + 1, bufs[1])

        @pl.when(k0 + 2 < NB)
        def _():
            start_batch(k0 + 2, bufs[0])

        acc = process_batch(p1, l1, acc)
        return acc

    acc = lax.fori_loop(0, NB // 2, body, zero)
    o_v[...] = acc
    pltpu.sync_copy(o_v, out_hbm.at[pl.ds(wid * L, L)])


@jax.jit
def _distractor_loss(p3, l3):
    mesh = plsc.VectorSubcoreMesh(core_axis_name="c", subcore_axis_name="s")
    partials = pl.kernel(
        _sc_body,
        mesh=mesh,
        out_type=jax.ShapeDtypeStruct((NW * L,), jnp.float32),
        scratch_types=[
            pltpu.VMEM((B_IMG, W, W), jnp.float32),
            pltpu.VMEM((B_IMG, W, W), jnp.float32),
            pltpu.VMEM((B_IMG, W, W), jnp.float32),
            pltpu.VMEM((B_IMG, W, W), jnp.float32),
            pltpu.VMEM((L,), jnp.float32),
            pltpu.SemaphoreType.DMA,
            pltpu.SemaphoreType.DMA,
        ],
        compiler_params=pltpu.CompilerParams(use_tc_tiling_on_sc=True),
    )(p3, l3)
    return partials.reshape(NW, L)[:, 0].sum() / (IMGS * K)


def kernel(prediction, label):
    p3 = prediction.reshape(IMGS, W, W)
    l3 = label.reshape(IMGS, W, W)
    return _distractor_loss(p3, l3)


# speedup vs baseline: 2.1927x; 1.2820x over previous
import functools

import jax
import jax.numpy as jnp
from jax import lax
from jax.experimental import pallas as pl
from jax.experimental.pallas import tpu as pltpu
from jax.experimental.pallas import tpu_sc as plsc

L = 16
NC = 2
NS = 16
NW = NC * NS
IMGS = 1024
IMG_N = 72 * 72
CHUNKS = IMG_N // L
IMGS_PER_W = IMGS // NW
B_IMG = 2
NB = IMGS_PER_W // B_IMG
UNROLL = 12
NSETS = 4
NEG_THRESHOLD = 0.01
K = 3


def _merge3(a, b):
    a1, a2, a3 = a
    b1, b2, b3 = b
    r1 = jnp.maximum(a1, b1)
    r2 = jnp.maximum(jnp.minimum(a1, b1), jnp.maximum(a2, b2))
    r3 = jnp.maximum(jnp.maximum(a3, b3),
                     jnp.maximum(jnp.minimum(a2, b1), jnp.minimum(a1, b2)))
    return r1, r2, r3


def _gather16(x, idx):
    return x.at[idx].get(mode="promise_in_bounds")


def _xlane_max(x, perms):
    for p in perms:
        x = jnp.maximum(x, _gather16(x, p))
    return x


def _xlane_min_i32(x, perms):
    for p in perms:
        x = jnp.minimum(x, _gather16(x, p))
    return x


def _sc_body(p_hbm, l_hbm, out_hbm, p0, l0, p1, l1, o_v, sem0, sem1):
    c = lax.axis_index("c")
    s = lax.axis_index("s")
    wid = s * NC + c
    base = wid * IMGS_PER_W

    lane = lax.broadcasted_iota(jnp.int32, (L,), 0)
    perms = [lane ^ (1 << k) for k in range(4)]
    zero = jnp.zeros((L,), jnp.float32)
    bufs = ((p0, l0, sem0), (p1, l1, sem1))

    def batch_copies(k, bufset):
        pb, lb, sem = bufset
        i0 = base + k * B_IMG
        return (pltpu.make_async_copy(p_hbm.at[pl.ds(i0, B_IMG)], pb, sem),
                pltpu.make_async_copy(l_hbm.at[pl.ds(i0, B_IMG)], lb, sem))

    def start_batch(k, bufset):
        for cp in batch_copies(k, bufset):
            cp.start()

    def wait_batch(k, bufset):
        for cp in batch_copies(k, bufset):
            cp.wait()

    def process_batch(pb, lb, acc):
        for i in range(B_IMG):

            def chunk_step(j, carry):
                sets = [list(carry[3 * s:3 * s + 3]) for s in range(NSETS)]
                for u in range(UNROLL):
                    t1, t2, t3 = sets[u % NSETS]
                    off = j * (UNROLL * L) + u * L
                    pv = pb[i, pl.ds(off, L)]
                    lv = lb[i, pl.ds(off, L)]
                    v = jnp.where(lv < NEG_THRESHOLD, pv, zero)
                    m1 = jnp.minimum(t1, v)
                    t1 = jnp.maximum(t1, v)
                    m2 = jnp.minimum(t2, m1)
                    t2 = jnp.maximum(t2, m1)
                    t3 = jnp.maximum(t3, m2)
                    sets[u % NSETS] = [t1, t2, t3]
                return tuple(x for s in sets for x in s)

            carry = lax.fori_loop(0, CHUNKS // UNROLL, chunk_step,
                                  (zero,) * (3 * NSETS))
            sets = [tuple(carry[3 * s:3 * s + 3]) for s in range(NSETS)]
            while len(sets) > 1:
                nxt = [_merge3(sets[k], sets[k + 1])
                       for k in range(0, len(sets) - 1, 2)]
                if len(sets) % 2:
                    nxt.append(sets[-1])
                sets = nxt
            t1, t2, t3 = sets[0]

            for k in range(K):
                g = _xlane_max(t1, perms)
                acc = acc + g
                if k == K - 1:
                    break
                idx = jnp.where(t1 == g, lane, L)
                mi = _xlane_min_i32(idx, perms)
                first = lane == mi
                t1 = jnp.where(first, t2, t1)
                t2 = jnp.where(first, t3, t2)
        return acc

    start_batch(0, bufs[0])

    def body(b, acc):
        k0 = 2 * b
        wait_batch(k0, bufs[0])
        start_batch(k0 + 1, bufs[1])
        acc = process_batch(p0, l0, acc)
        wait_batch(k0 + 1, bufs[1])

        @pl.when(k0 + 2 < NB)
        def _():
            start_batch(k0 + 2, bufs[0])

        acc = process_batch(p1, l1, acc)
        return acc

    acc = lax.fori_loop(0, NB // 2, body, zero)
    o_v[...] = acc
    pltpu.sync_copy(o_v, out_hbm.at[pl.ds(wid * L, L)])


@jax.jit
def _distractor_loss(p3, l3):
    mesh = plsc.VectorSubcoreMesh(core_axis_name="c", subcore_axis_name="s")
    partials = pl.kernel(
        _sc_body,
        mesh=mesh,
        out_type=jax.ShapeDtypeStruct((NW * L,), jnp.float32),
        scratch_types=[
            pltpu.VMEM((B_IMG, IMG_N), jnp.float32),
            pltpu.VMEM((B_IMG, IMG_N), jnp.float32),
            pltpu.VMEM((B_IMG, IMG_N), jnp.float32),
            pltpu.VMEM((B_IMG, IMG_N), jnp.float32),
            pltpu.VMEM((L,), jnp.float32),
            pltpu.SemaphoreType.DMA,
            pltpu.SemaphoreType.DMA,
        ],
        compiler_params=pltpu.CompilerParams(use_tc_tiling_on_sc=True),
    )(p3, l3)
    return partials.reshape(NW, L)[:, 0].sum() / (IMGS * K)


def kernel(prediction, label):
    p3 = prediction.reshape(IMGS, IMG_N)
    l3 = label.reshape(IMGS, IMG_N)
    return _distractor_loss(p3, l3)


# speedup vs baseline: 2.1943x; 1.0007x over previous
import functools

import jax
import jax.numpy as jnp
from jax import lax
from jax.experimental import pallas as pl
from jax.experimental.pallas import tpu as pltpu
from jax.experimental.pallas import tpu_sc as plsc

L = 16
NC = 2
NS = 16
NW = NC * NS
IMGS = 1024
IMG_N = 72 * 72
CHUNKS = IMG_N // L
IMGS_PER_W = IMGS // NW
B_IMG = 2
NB = IMGS_PER_W // B_IMG
UNROLL = 12
NSETS = 6
NEG_THRESHOLD = 0.01
K = 3


def _merge3(a, b):
    a1, a2, a3 = a
    b1, b2, b3 = b
    r1 = jnp.maximum(a1, b1)
    r2 = jnp.maximum(jnp.minimum(a1, b1), jnp.maximum(a2, b2))
    r3 = jnp.maximum(jnp.maximum(a3, b3),
                     jnp.maximum(jnp.minimum(a2, b1), jnp.minimum(a1, b2)))
    return r1, r2, r3


def _gather16(x, idx):
    return x.at[idx].get(mode="promise_in_bounds")


def _xlane_max(x, perms):
    for p in perms:
        x = jnp.maximum(x, _gather16(x, p))
    return x


def _xlane_min_i32(x, perms):
    for p in perms:
        x = jnp.minimum(x, _gather16(x, p))
    return x


def _sc_body(p_hbm, l_hbm, out_hbm, p0, l0, p1, l1, o_v, sem0, sem1):
    c = lax.axis_index("c")
    s = lax.axis_index("s")
    wid = s * NC + c
    base = wid * IMGS_PER_W

    lane = lax.broadcasted_iota(jnp.int32, (L,), 0)
    perms = [lane ^ (1 << k) for k in range(4)]
    zero = jnp.zeros((L,), jnp.float32)
    bufs = ((p0, l0, sem0), (p1, l1, sem1))

    def batch_copies(k, bufset):
        pb, lb, sem = bufset
        i0 = base + k * B_IMG
        return (pltpu.make_async_copy(p_hbm.at[pl.ds(i0, B_IMG)], pb, sem),
                pltpu.make_async_copy(l_hbm.at[pl.ds(i0, B_IMG)], lb, sem))

    def start_batch(k, bufset):
        for cp in batch_copies(k, bufset):
            cp.start()

    def wait_batch(k, bufset):
        for cp in batch_copies(k, bufset):
            cp.wait()

    def process_batch(pb, lb, acc):
        for i in range(B_IMG):

            def chunk_step(j, carry):
                sets = [list(carry[4 * s:4 * s + 4]) for s in range(NSETS)]
                for u in range(UNROLL // 2):
                    t1, t2, t3, b1 = sets[u % NSETS]
                    off = j * (UNROLL * L) + 2 * u * L
                    pa = pb[i, pl.ds(off, L)]
                    la = lb[i, pl.ds(off, L)]
                    pc = pb[i, pl.ds(off + L, L)]
                    lc = lb[i, pl.ds(off + L, L)]
                    va = jnp.where(la < NEG_THRESHOLD, pa, zero)
                    vc = jnp.where(lc < NEG_THRESHOLD, pc, zero)
                    hi = jnp.maximum(va, vc)
                    lo = jnp.minimum(va, vc)
                    m1 = jnp.minimum(t1, hi)
                    t1 = jnp.maximum(t1, hi)
                    m2 = jnp.minimum(t2, m1)
                    t2 = jnp.maximum(t2, m1)
                    t3 = jnp.maximum(t3, m2)
                    b1 = jnp.maximum(b1, lo)
                    sets[u % NSETS] = [t1, t2, t3, b1]
                return tuple(x for s in sets for x in s)

            carry = lax.fori_loop(0, CHUNKS // UNROLL, chunk_step,
                                  (zero,) * (4 * NSETS))
            sets = []
            for s in range(NSETS):
                t1, t2, t3, b1 = carry[4 * s:4 * s + 4]
                r1 = jnp.maximum(t1, b1)
                r2 = jnp.maximum(jnp.minimum(t1, b1), t2)
                r3 = jnp.maximum(t3, jnp.minimum(t2, b1))
                sets.append((r1, r2, r3))
            while len(sets) > 1:
                nxt = [_merge3(sets[k], sets[k + 1])
                       for k in range(0, len(sets) - 1, 2)]
                if len(sets) % 2:
                    nxt.append(sets[-1])
                sets = nxt
            t1, t2, t3 = sets[0]

            for k in range(K):
                g = _xlane_max(t1, perms)
                acc = acc + g
                if k == K - 1:
                    break
                idx = jnp.where(t1 == g, lane, L)
                mi = _xlane_min_i32(idx, perms)
                first = lane == mi
                t1 = jnp.where(first, t2, t1)
                t2 = jnp.where(first, t3, t2)
        return acc

    start_batch(0, bufs[0])

    def body(b, acc):
        k0 = 2 * b
        wait_batch(k0, bufs[0])
        start_batch(k0 + 1, bufs[1])
        acc = process_batch(p0, l0, acc)
        wait_batch(k0 + 1, bufs[1])

        @pl.when(k0 + 2 < NB)
        def _():
            start_batch(k0 + 2, bufs[0])

        acc = process_batch(p1, l1, acc)
        return acc

    acc = lax.fori_loop(0, NB // 2, body, zero)
    o_v[...] = acc
    pltpu.sync_copy(o_v, out_hbm.at[pl.ds(wid * L, L)])


@jax.jit
def _distractor_loss(p3, l3):
    mesh = plsc.VectorSubcoreMesh(core_axis_name="c", subcore_axis_name="s")
    partials = pl.kernel(
        _sc_body,
        mesh=mesh,
        out_type=jax.ShapeDtypeStruct((NW * L,), jnp.float32),
        scratch_types=[
            pltpu.VMEM((B_IMG, IMG_N), jnp.float32),
            pltpu.VMEM((B_IMG, IMG_N), jnp.float32),
            pltpu.VMEM((B_IMG, IMG_N), jnp.float32),
            pltpu.VMEM((B_IMG, IMG_N), jnp.float32),
            pltpu.VMEM((L,), jnp.float32),
            pltpu.SemaphoreType.DMA,
            pltpu.SemaphoreType.DMA,
        ],
        compiler_params=pltpu.CompilerParams(use_tc_tiling_on_sc=True),
    )(p3, l3)
    return partials.reshape(NW, L)[:, 0].sum() / (IMGS * K)


def kernel(prediction, label):
    p3 = prediction.reshape(IMGS, IMG_N)
    l3 = label.reshape(IMGS, IMG_N)
    return _distractor_loss(p3, l3)


# speedup vs baseline: 2.2882x; 1.0428x over previous
import functools

import jax
import jax.numpy as jnp
from jax import lax
from jax.experimental import pallas as pl
from jax.experimental.pallas import tpu as pltpu
from jax.experimental.pallas import tpu_sc as plsc

L = 16
NC = 2
NS = 16
NW = NC * NS
IMGS = 1024
IMG_N = 72 * 72
CHUNKS = IMG_N // L
IMGS_PER_W = IMGS // NW
B_IMG = 4
NB = IMGS_PER_W // B_IMG
UNROLL = 12
NSETS = 6
NEG_THRESHOLD = 0.01
K = 3


def _merge3(a, b):
    a1, a2, a3 = a
    b1, b2, b3 = b
    r1 = jnp.maximum(a1, b1)
    r2 = jnp.maximum(jnp.minimum(a1, b1), jnp.maximum(a2, b2))
    r3 = jnp.maximum(jnp.maximum(a3, b3),
                     jnp.maximum(jnp.minimum(a2, b1), jnp.minimum(a1, b2)))
    return r1, r2, r3


def _gather16(x, idx):
    return x.at[idx].get(mode="promise_in_bounds")


def _xlane_max(x, perms):
    for p in perms:
        x = jnp.maximum(x, _gather16(x, p))
    return x


def _xlane_min_i32(x, perms):
    for p in perms:
        x = jnp.minimum(x, _gather16(x, p))
    return x


def _sc_body(p_hbm, l_hbm, out_hbm, p0, l0, p1, l1, o_v, sem0, sem1):
    c = lax.axis_index("c")
    s = lax.axis_index("s")
    wid = s * NC + c
    base = wid * IMGS_PER_W

    lane = lax.broadcasted_iota(jnp.int32, (L,), 0)
    perms = [lane ^ (1 << k) for k in range(4)]
    zero = jnp.zeros((L,), jnp.float32)
    bufs = ((p0, l0, sem0), (p1, l1, sem1))

    def batch_copies(k, bufset):
        pb, lb, sem = bufset
        i0 = base + k * B_IMG
        return (pltpu.make_async_copy(p_hbm.at[pl.ds(i0, B_IMG)], pb, sem),
                pltpu.make_async_copy(l_hbm.at[pl.ds(i0, B_IMG)], lb, sem))

    def start_batch(k, bufset):
        for cp in batch_copies(k, bufset):
            cp.start()

    def wait_batch(k, bufset):
        for cp in batch_copies(k, bufset):
            cp.wait()

    def process_batch(pb, lb, acc):
        for i in range(B_IMG):

            def chunk_step(j, carry):
                sets = [list(carry[4 * s:4 * s + 4]) for s in range(NSETS)]
                for u in range(UNROLL // 2):
                    t1, t2, t3, b1 = sets[u % NSETS]
                    off = j * (UNROLL * L) + 2 * u * L
                    pa = pb[i, pl.ds(off, L)]
                    la = lb[i, pl.ds(off, L)]
                    pc = pb[i, pl.ds(off + L, L)]
                    lc = lb[i, pl.ds(off + L, L)]
                    va = jnp.where(la < NEG_THRESHOLD, pa, zero)
                    vc = jnp.where(lc < NEG_THRESHOLD, pc, zero)
                    hi = jnp.maximum(va, vc)
                    lo = jnp.minimum(va, vc)
                    m1 = jnp.minimum(t1, hi)
                    t1 = jnp.maximum(t1, hi)
                    m2 = jnp.minimum(t2, m1)
                    t2 = jnp.maximum(t2, m1)
                    t3 = jnp.maximum(t3, m2)
                    b1 = jnp.maximum(b1, lo)
                    sets[u % NSETS] = [t1, t2, t3, b1]
                return tuple(x for s in sets for x in s)

            carry = lax.fori_loop(0, CHUNKS // UNROLL, chunk_step,
                                  (zero,) * (4 * NSETS))
            sets = []
            for s in range(NSETS):
                t1, t2, t3, b1 = carry[4 * s:4 * s + 4]
                r1 = jnp.maximum(t1, b1)
                r2 = jnp.maximum(jnp.minimum(t1, b1), t2)
                r3 = jnp.maximum(t3, jnp.minimum(t2, b1))
                sets.append((r1, r2, r3))
            while len(sets) > 1:
                nxt = [_merge3(sets[k], sets[k + 1])
                       for k in range(0, len(sets) - 1, 2)]
                if len(sets) % 2:
                    nxt.append(sets[-1])
                sets = nxt
            t1, t2, t3 = sets[0]

            for k in range(K):
                g = _xlane_max(t1, perms)
                acc = acc + g
                if k == K - 1:
                    break
                idx = jnp.where(t1 == g, lane, L)
                mi = _xlane_min_i32(idx, perms)
                first = lane == mi
                t1 = jnp.where(first, t2, t1)
                t2 = jnp.where(first, t3, t2)
        return acc

    start_batch(0, bufs[0])

    def body(b, acc):
        k0 = 2 * b
        wait_batch(k0, bufs[0])
        start_batch(k0 + 1, bufs[1])
        acc = process_batch(p0, l0, acc)
        wait_batch(k0 + 1, bufs[1])

        @pl.when(k0 + 2 < NB)
        def _():
            start_batch(k0 + 2, bufs[0])

        acc = process_batch(p1, l1, acc)
        return acc

    acc = lax.fori_loop(0, NB // 2, body, zero)
    o_v[...] = acc
    pltpu.sync_copy(o_v, out_hbm.at[pl.ds(wid * L, L)])


@jax.jit
def _distractor_loss(p3, l3):
    mesh = plsc.VectorSubcoreMesh(core_axis_name="c", subcore_axis_name="s")
    partials = pl.kernel(
        _sc_body,
        mesh=mesh,
        out_type=jax.ShapeDtypeStruct((NW * L,), jnp.float32),
        scratch_types=[
            pltpu.VMEM((B_IMG, IMG_N), jnp.float32),
            pltpu.VMEM((B_IMG, IMG_N), jnp.float32),
            pltpu.VMEM((B_IMG, IMG_N), jnp.float32),
            pltpu.VMEM((B_IMG, IMG_N), jnp.float32),
            pltpu.VMEM((L,), jnp.float32),
            pltpu.SemaphoreType.DMA,
            pltpu.SemaphoreType.DMA,
        ],
        compiler_params=pltpu.CompilerParams(use_tc_tiling_on_sc=True),
    )(p3, l3)
    return partials.reshape(NW, L)[:, 0].sum() / (IMGS * K)


def kernel(prediction, label):
    p3 = prediction.reshape(IMGS, IMG_N)
    l3 = label.reshape(IMGS, IMG_N)
    return _distractor_loss(p3, l3)
